# Initial kernel scaffold; baseline (speedup 1.0000x reference)
import functools

import jax
import jax.numpy as jnp
from jax import lax
from jax.experimental import pallas as pl
from jax.experimental.pallas import tpu as pltpu
from jax.experimental.pallas import tpu_sc as plsc

N = 10000
D = 128
E = 320000
NC, NS = 2, 16
NW = NC * NS
K = 128
EPT = 10240
E_PAD = NW * EPT
CH = EPT // K
RPT = N // NS
N_PAD = N + 16
SINK = N
TB = 1000


def _vector_mesh():
    return plsc.VectorSubcoreMesh(core_axis_name="c", subcore_axis_name="s")


def _dis_block(da_ref, db_ref):
    deg = da_ref[:, 0:1] + db_ref[:, 0:1] + 1.0
    return lax.rsqrt(deg)


def _tc_matmul(x, w):
    def body(x_ref, w_ref, o_ref):
        o_ref[...] = jnp.dot(x_ref[...], w_ref[...],
                             preferred_element_type=jnp.float32)
    return pl.pallas_call(
        body,
        grid=(N // TB,),
        in_specs=[pl.BlockSpec((TB, D), lambda i: (i, 0)),
                  pl.BlockSpec((D, D), lambda i: (0, 0))],
        out_specs=pl.BlockSpec((TB, D), lambda i: (i, 0)),
        out_shape=jax.ShapeDtypeStruct((N, D), jnp.float32),
    )(x, w)


def _tc_scale(xw, deg_a, deg_b):
    def body(x_ref, da_ref, db_ref, o_ref):
        o_ref[...] = x_ref[...] * _dis_block(da_ref, db_ref)
    return pl.pallas_call(
        body,
        grid=(N // TB,),
        in_specs=[pl.BlockSpec((TB, D), lambda i: (i, 0)),
                  pl.BlockSpec((TB, 16), lambda i: (i, 0)),
                  pl.BlockSpec((TB, 16), lambda i: (i, 0))],
        out_specs=pl.BlockSpec((TB, D), lambda i: (i, 0)),
        out_shape=jax.ShapeDtypeStruct((N, D), jnp.float32),
    )(xw, deg_a, deg_b)


def _tc_mid(acc_a, acc_b, deg_a, deg_b, b1, w2):
    def body(aa_ref, ab_ref, da_ref, db_ref, b_ref, w_ref, o_ref):
        dis = _dis_block(da_ref, db_ref)
        h = jnp.maximum(dis * (aa_ref[...] + ab_ref[...]) + b_ref[...], 0.0)
        o_ref[...] = jnp.dot(h, w_ref[...],
                             preferred_element_type=jnp.float32) * dis
    return pl.pallas_call(
        body,
        grid=(N // TB,),
        in_specs=[pl.BlockSpec((TB, D), lambda i: (i, 0)),
                  pl.BlockSpec((TB, D), lambda i: (i, 0)),
                  pl.BlockSpec((TB, 16), lambda i: (i, 0)),
                  pl.BlockSpec((TB, 16), lambda i: (i, 0)),
                  pl.BlockSpec((1, D), lambda i: (0, 0)),
                  pl.BlockSpec((D, D), lambda i: (0, 0))],
        out_specs=pl.BlockSpec((TB, D), lambda i: (i, 0)),
        out_shape=jax.ShapeDtypeStruct((N, D), jnp.float32),
    )(acc_a, acc_b, deg_a, deg_b, b1.reshape(1, D), w2)


def _tc_final(acc_a, acc_b, deg_a, deg_b, b2):
    def body(aa_ref, ab_ref, da_ref, db_ref, b_ref, o_ref):
        dis = _dis_block(da_ref, db_ref)
        o_ref[...] = dis * (aa_ref[...] + ab_ref[...]) + b_ref[...]
    return pl.pallas_call(
        body,
        grid=(N // TB,),
        in_specs=[pl.BlockSpec((TB, D), lambda i: (i, 0)),
                  pl.BlockSpec((TB, D), lambda i: (i, 0)),
                  pl.BlockSpec((TB, 16), lambda i: (i, 0)),
                  pl.BlockSpec((TB, 16), lambda i: (i, 0)),
                  pl.BlockSpec((1, D), lambda i: (0, 0))],
        out_specs=pl.BlockSpec((TB, D), lambda i: (i, 0)),
        out_shape=jax.ShapeDtypeStruct((N, D), jnp.float32),
    )(acc_a, acc_b, deg_a, deg_b, b2.reshape(1, D))


def _sc_degree(dst, zeros16, ones16):
    @functools.partial(
        pl.kernel,
        out_type=[jax.ShapeDtypeStruct((N, 16), jnp.float32),
                  jax.ShapeDtypeStruct((N, 16), jnp.float32)],
        mesh=_vector_mesh(),
        scratch_types=[
            pltpu.VMEM_SHARED((N_PAD, 16), jnp.float32),
            pltpu.VMEM((K,), jnp.int32),
            pltpu.VMEM((K, 16), jnp.float32),
            pltpu.SemaphoreType.DMA,
        ],
    )
    def deg_kernel(dst_hbm, z_hbm, ones_hbm, dega_hbm, degb_hbm,
                   deg_sh, idx_v, ones_v, sem):
        c = lax.axis_index("c")
        s = lax.axis_index("s")
        pltpu.sync_copy(z_hbm, deg_sh.at[pl.ds(s * RPT, RPT)])
        pltpu.sync_copy(ones_hbm, ones_v)
        plsc.subcore_barrier()
        base = (c * NS + s) * EPT

        @pl.loop(0, CH)
        def _(g):
            pltpu.sync_copy(dst_hbm.at[pl.ds(base + g * K, K)], idx_v)
            pltpu.sync_copy(ones_v, deg_sh.at[idx_v], add=True)

        plsc.subcore_barrier()

        @pl.when(c == 0)
        def _():
            pltpu.sync_copy(deg_sh.at[pl.ds(s * RPT, RPT)],
                            dega_hbm.at[pl.ds(s * RPT, RPT)])

        @pl.when(c == 1)
        def _():
            pltpu.sync_copy(deg_sh.at[pl.ds(s * RPT, RPT)],
                            degb_hbm.at[pl.ds(s * RPT, RPT)])

    return deg_kernel(dst, zeros16, ones16)


def _sc_gather_scatter(y, src, dst, zeros128):
    @functools.partial(
        pl.kernel,
        out_type=[jax.ShapeDtypeStruct((N, D), jnp.float32),
                  jax.ShapeDtypeStruct((N, D), jnp.float32)],
        mesh=_vector_mesh(),
        scratch_types=[
            pltpu.VMEM_SHARED((N_PAD, D), jnp.float32),
            pltpu.VMEM((K,), jnp.int32),
            pltpu.VMEM((K,), jnp.int32),
            pltpu.VMEM((K, D), jnp.float32),
            pltpu.SemaphoreType.DMA,
        ],
    )
    def gs_kernel(y_hbm, src_hbm, dst_hbm, z_hbm, acca_hbm, accb_hbm,
                  acc_sh, sidx_v, didx_v, buf_v, gsem):
        c = lax.axis_index("c")
        s = lax.axis_index("s")

        @pl.when(c == 0)
        def _():
            pltpu.sync_copy(y_hbm.at[pl.ds(s * RPT, RPT)],
                            acc_sh.at[pl.ds(s * RPT, RPT)])

        @pl.when(c == 1)
        def _():
            pltpu.sync_copy(z_hbm, acc_sh.at[pl.ds(s * RPT, RPT)])

        plsc.subcore_barrier()
        base = (c * NS + s) * EPT

        @pl.loop(0, CH)
        def _(g):
            off = base + g * K
            pltpu.sync_copy(src_hbm.at[pl.ds(off, K)], sidx_v)
            pltpu.sync_copy(dst_hbm.at[pl.ds(off, K)], didx_v)
            pltpu.async_copy(y_hbm.at[sidx_v], buf_v, gsem).wait()
            pltpu.sync_copy(buf_v, acc_sh.at[didx_v], add=True)

        plsc.subcore_barrier()

        @pl.when(c == 0)
        def _():
            pltpu.sync_copy(acc_sh.at[pl.ds(s * RPT, RPT)],
                            acca_hbm.at[pl.ds(s * RPT, RPT)])

        @pl.when(c == 1)
        def _():
            pltpu.sync_copy(acc_sh.at[pl.ds(s * RPT, RPT)],
                            accb_hbm.at[pl.ds(s * RPT, RPT)])

    return gs_kernel(y, src, dst, zeros128)


def kernel(x, edge_index, W1, b1, W2, b2):
    ei = edge_index.astype(jnp.int32)
    npad = E_PAD - E
    src = jnp.concatenate([ei[0], jnp.zeros((npad,), jnp.int32)])
    dst = jnp.concatenate([ei[1], jnp.full((npad,), SINK, jnp.int32)])
    zeros16 = jnp.zeros((RPT, 16), jnp.float32)
    ones16 = jnp.ones((K, 16), jnp.float32)
    zeros128 = jnp.zeros((RPT, D), jnp.float32)

    xw1 = _tc_matmul(x, W1)
    deg_a, deg_b = _sc_degree(dst, zeros16, ones16)
    y1 = _tc_scale(xw1, deg_a, deg_b)
    acc_a1, acc_b1 = _sc_gather_scatter(y1, src, dst, zeros128)
    y2 = _tc_mid(acc_a1, acc_b1, deg_a, deg_b, b1, W2)
    acc_a2, acc_b2 = _sc_gather_scatter(y2, src, dst, zeros128)
    return _tc_final(acc_a2, acc_b2, deg_a, deg_b, b2)


# speedup vs baseline: 7.8888x; 7.8888x over previous
import functools

import jax
import jax.numpy as jnp
from jax import lax
from jax.experimental import pallas as pl
from jax.experimental.pallas import tpu as pltpu
from jax.experimental.pallas import tpu_sc as plsc

N = 10000
D = 128
E = 320000
NC, NS = 2, 16
NW = NC * NS
K = 128
EPT = 10240
E_PAD = NW * EPT
CH = EPT // K
RPT = N // NS
N_PAD = N + 16
SINK = N
TB = 1000


def _vector_mesh():
    return plsc.VectorSubcoreMesh(core_axis_name="c", subcore_axis_name="s")


_SC_PARAMS = pltpu.CompilerParams(use_tc_tiling_on_sc=False)


def _dis_block(da_ref, db_ref):
    deg = da_ref[:, 0:1] + db_ref[:, 0:1] + 1.0
    return lax.rsqrt(deg)


def _tc_matmul(x, w):
    def body(x_ref, w_ref, o_ref):
        o_ref[...] = jnp.dot(x_ref[...], w_ref[...],
                             preferred_element_type=jnp.float32)
    return pl.pallas_call(
        body,
        grid=(N // TB,),
        in_specs=[pl.BlockSpec((TB, D), lambda i: (i, 0)),
                  pl.BlockSpec((D, D), lambda i: (0, 0))],
        out_specs=pl.BlockSpec((TB, D), lambda i: (i, 0)),
        out_shape=jax.ShapeDtypeStruct((N, D), jnp.float32),
    )(x, w)


def _tc_scale(xw, deg_a, deg_b):
    def body(x_ref, da_ref, db_ref, o_ref):
        o_ref[...] = x_ref[...] * _dis_block(da_ref, db_ref)
    return pl.pallas_call(
        body,
        grid=(N // TB,),
        in_specs=[pl.BlockSpec((TB, D), lambda i: (i, 0)),
                  pl.BlockSpec((TB, 16), lambda i: (i, 0)),
                  pl.BlockSpec((TB, 16), lambda i: (i, 0))],
        out_specs=pl.BlockSpec((TB, D), lambda i: (i, 0)),
        out_shape=jax.ShapeDtypeStruct((N, D), jnp.float32),
    )(xw, deg_a, deg_b)


def _tc_mid(acc_a, acc_b, deg_a, deg_b, b1, w2):
    def body(aa_ref, ab_ref, da_ref, db_ref, b_ref, w_ref, o_ref):
        dis = _dis_block(da_ref, db_ref)
        h = jnp.maximum(dis * (aa_ref[...] + ab_ref[...]) + b_ref[...], 0.0)
        o_ref[...] = jnp.dot(h, w_ref[...],
                             preferred_element_type=jnp.float32) * dis
    return pl.pallas_call(
        body,
        grid=(N // TB,),
        in_specs=[pl.BlockSpec((TB, D), lambda i: (i, 0)),
                  pl.BlockSpec((TB, D), lambda i: (i, 0)),
                  pl.BlockSpec((TB, 16), lambda i: (i, 0)),
                  pl.BlockSpec((TB, 16), lambda i: (i, 0)),
                  pl.BlockSpec((1, D), lambda i: (0, 0)),
                  pl.BlockSpec((D, D), lambda i: (0, 0))],
        out_specs=pl.BlockSpec((TB, D), lambda i: (i, 0)),
        out_shape=jax.ShapeDtypeStruct((N, D), jnp.float32),
    )(acc_a, acc_b, deg_a, deg_b, b1.reshape(1, D), w2)


def _tc_final(acc_a, acc_b, deg_a, deg_b, b2):
    def body(aa_ref, ab_ref, da_ref, db_ref, b_ref, o_ref):
        dis = _dis_block(da_ref, db_ref)
        o_ref[...] = dis * (aa_ref[...] + ab_ref[...]) + b_ref[...]
    return pl.pallas_call(
        body,
        grid=(N // TB,),
        in_specs=[pl.BlockSpec((TB, D), lambda i: (i, 0)),
                  pl.BlockSpec((TB, D), lambda i: (i, 0)),
                  pl.BlockSpec((TB, 16), lambda i: (i, 0)),
                  pl.BlockSpec((TB, 16), lambda i: (i, 0)),
                  pl.BlockSpec((1, D), lambda i: (0, 0))],
        out_specs=pl.BlockSpec((TB, D), lambda i: (i, 0)),
        out_shape=jax.ShapeDtypeStruct((N, D), jnp.float32),
    )(acc_a, acc_b, deg_a, deg_b, b2.reshape(1, D))


def _sc_degree(dst, zeros16, ones16):
    @functools.partial(
        pl.kernel,
        out_type=[jax.ShapeDtypeStruct((N, 16), jnp.float32),
                  jax.ShapeDtypeStruct((N, 16), jnp.float32)],
        mesh=_vector_mesh(),
        scratch_types=[
            pltpu.VMEM_SHARED((N_PAD, 16), jnp.float32),
            pltpu.VMEM((K,), jnp.int32),
            pltpu.VMEM((K, 16), jnp.float32),
            pltpu.SemaphoreType.DMA,
        ],
        compiler_params=_SC_PARAMS,
    )
    def deg_kernel(dst_hbm, z_hbm, ones_hbm, dega_hbm, degb_hbm,
                   deg_sh, idx_v, ones_v, sem):
        c = lax.axis_index("c")
        s = lax.axis_index("s")
        pltpu.sync_copy(z_hbm, deg_sh.at[pl.ds(s * RPT, RPT)])
        pltpu.sync_copy(ones_hbm, ones_v)
        plsc.subcore_barrier()
        base = (c * NS + s) * EPT

        @pl.loop(0, CH)
        def _(g):
            pltpu.sync_copy(dst_hbm.at[pl.ds(base + g * K, K)], idx_v)
            pltpu.sync_copy(ones_v, deg_sh.at[idx_v], add=True)

        plsc.subcore_barrier()

        @pl.when(c == 0)
        def _():
            pltpu.sync_copy(deg_sh.at[pl.ds(s * RPT, RPT)],
                            dega_hbm.at[pl.ds(s * RPT, RPT)])

        @pl.when(c == 1)
        def _():
            pltpu.sync_copy(deg_sh.at[pl.ds(s * RPT, RPT)],
                            degb_hbm.at[pl.ds(s * RPT, RPT)])

    return deg_kernel(dst, zeros16, ones16)


def _sc_gather_scatter(y, src, dst, zeros128):
    @functools.partial(
        pl.kernel,
        out_type=[jax.ShapeDtypeStruct((N, D), jnp.float32),
                  jax.ShapeDtypeStruct((N, D), jnp.float32)],
        mesh=_vector_mesh(),
        scratch_types=[
            pltpu.VMEM_SHARED((N_PAD, D), jnp.float32),
            pltpu.VMEM((K,), jnp.int32),
            pltpu.VMEM((K,), jnp.int32),
            pltpu.VMEM((K, D), jnp.float32),
            pltpu.SemaphoreType.DMA,
        ],
        compiler_params=_SC_PARAMS,
    )
    def gs_kernel(y_hbm, src_hbm, dst_hbm, z_hbm, acca_hbm, accb_hbm,
                  acc_sh, sidx_v, didx_v, buf_v, gsem):
        c = lax.axis_index("c")
        s = lax.axis_index("s")

        @pl.when(c == 0)
        def _():
            pltpu.sync_copy(y_hbm.at[pl.ds(s * RPT, RPT)],
                            acc_sh.at[pl.ds(s * RPT, RPT)])

        @pl.when(c == 1)
        def _():
            pltpu.sync_copy(z_hbm, acc_sh.at[pl.ds(s * RPT, RPT)])

        plsc.subcore_barrier()
        base = (c * NS + s) * EPT

        @pl.loop(0, CH)
        def _(g):
            off = base + g * K
            pltpu.sync_copy(src_hbm.at[pl.ds(off, K)], sidx_v)
            pltpu.sync_copy(dst_hbm.at[pl.ds(off, K)], didx_v)
            pltpu.async_copy(y_hbm.at[sidx_v], buf_v, gsem).wait()
            pltpu.sync_copy(buf_v, acc_sh.at[didx_v], add=True)

        plsc.subcore_barrier()

        @pl.when(c == 0)
        def _():
            pltpu.sync_copy(acc_sh.at[pl.ds(s * RPT, RPT)],
                            acca_hbm.at[pl.ds(s * RPT, RPT)])

        @pl.when(c == 1)
        def _():
            pltpu.sync_copy(acc_sh.at[pl.ds(s * RPT, RPT)],
                            accb_hbm.at[pl.ds(s * RPT, RPT)])

    return gs_kernel(y, src, dst, zeros128)


def kernel(x, edge_index, W1, b1, W2, b2):
    ei = edge_index.astype(jnp.int32)
    npad = E_PAD - E
    src = jnp.concatenate([ei[0], jnp.zeros((npad,), jnp.int32)])
    dst = jnp.concatenate([ei[1], jnp.full((npad,), SINK, jnp.int32)])
    zeros16 = jnp.zeros((RPT, 16), jnp.float32)
    ones16 = jnp.ones((K, 16), jnp.float32)
    zeros128 = jnp.zeros((RPT, D), jnp.float32)

    xw1 = _tc_matmul(x, W1)
    deg_a, deg_b = _sc_degree(dst, zeros16, ones16)
    y1 = _tc_scale(xw1, deg_a, deg_b)
    acc_a1, acc_b1 = _sc_gather_scatter(y1, src, dst, zeros128)
    y2 = _tc_mid(acc_a1, acc_b1, deg_a, deg_b, b1, W2)
    acc_a2, acc_b2 = _sc_gather_scatter(y2, src, dst, zeros128)
    return _tc_final(acc_a2, acc_b2, deg_a, deg_b, b2)


# speedup vs baseline: 8.4453x; 1.0705x over previous
import functools

import jax
import jax.numpy as jnp
from jax import lax
from jax.experimental import pallas as pl
from jax.experimental.pallas import tpu as pltpu
from jax.experimental.pallas import tpu_sc as plsc

N = 10000
D = 128
E = 320000
NC, NS = 2, 16
NW = NC * NS
K = 128
EPT = 10240
E_PAD = NW * EPT
CH = EPT // K
RPT = N // NS
N_PAD = N + 16
SINK = N
TB = 1000


def _vector_mesh():
    return plsc.VectorSubcoreMesh(core_axis_name="c", subcore_axis_name="s")


_SC_PARAMS = pltpu.CompilerParams(use_tc_tiling_on_sc=False)


def _dis_block(da_ref, db_ref):
    deg = da_ref[:, 0:1] + db_ref[:, 0:1] + 1.0
    return lax.rsqrt(deg)


def _tc_matmul(x, w):
    def body(x_ref, w_ref, o_ref):
        o_ref[...] = jnp.dot(x_ref[...], w_ref[...],
                             preferred_element_type=jnp.float32)
    return pl.pallas_call(
        body,
        grid=(N // TB,),
        in_specs=[pl.BlockSpec((TB, D), lambda i: (i, 0)),
                  pl.BlockSpec((D, D), lambda i: (0, 0))],
        out_specs=pl.BlockSpec((TB, D), lambda i: (i, 0)),
        out_shape=jax.ShapeDtypeStruct((N, D), jnp.float32),
    )(x, w)


def _tc_scale(xw, deg_a, deg_b):
    def body(x_ref, da_ref, db_ref, o_ref):
        o_ref[...] = x_ref[...] * _dis_block(da_ref, db_ref)
    return pl.pallas_call(
        body,
        grid=(N // TB,),
        in_specs=[pl.BlockSpec((TB, D), lambda i: (i, 0)),
                  pl.BlockSpec((TB, 16), lambda i: (i, 0)),
                  pl.BlockSpec((TB, 16), lambda i: (i, 0))],
        out_specs=pl.BlockSpec((TB, D), lambda i: (i, 0)),
        out_shape=jax.ShapeDtypeStruct((N, D), jnp.float32),
    )(xw, deg_a, deg_b)


def _tc_mid(acc_a, acc_b, deg_a, deg_b, b1, w2):
    def body(aa_ref, ab_ref, da_ref, db_ref, b_ref, w_ref, o_ref):
        dis = _dis_block(da_ref, db_ref)
        h = jnp.maximum(dis * (aa_ref[...] + ab_ref[...]) + b_ref[...], 0.0)
        o_ref[...] = jnp.dot(h, w_ref[...],
                             preferred_element_type=jnp.float32) * dis
    return pl.pallas_call(
        body,
        grid=(N // TB,),
        in_specs=[pl.BlockSpec((TB, D), lambda i: (i, 0)),
                  pl.BlockSpec((TB, D), lambda i: (i, 0)),
                  pl.BlockSpec((TB, 16), lambda i: (i, 0)),
                  pl.BlockSpec((TB, 16), lambda i: (i, 0)),
                  pl.BlockSpec((1, D), lambda i: (0, 0)),
                  pl.BlockSpec((D, D), lambda i: (0, 0))],
        out_specs=pl.BlockSpec((TB, D), lambda i: (i, 0)),
        out_shape=jax.ShapeDtypeStruct((N, D), jnp.float32),
    )(acc_a, acc_b, deg_a, deg_b, b1.reshape(1, D), w2)


def _tc_final(acc_a, acc_b, deg_a, deg_b, b2):
    def body(aa_ref, ab_ref, da_ref, db_ref, b_ref, o_ref):
        dis = _dis_block(da_ref, db_ref)
        o_ref[...] = dis * (aa_ref[...] + ab_ref[...]) + b_ref[...]
    return pl.pallas_call(
        body,
        grid=(N // TB,),
        in_specs=[pl.BlockSpec((TB, D), lambda i: (i, 0)),
                  pl.BlockSpec((TB, D), lambda i: (i, 0)),
                  pl.BlockSpec((TB, 16), lambda i: (i, 0)),
                  pl.BlockSpec((TB, 16), lambda i: (i, 0)),
                  pl.BlockSpec((1, D), lambda i: (0, 0))],
        out_specs=pl.BlockSpec((TB, D), lambda i: (i, 0)),
        out_shape=jax.ShapeDtypeStruct((N, D), jnp.float32),
    )(acc_a, acc_b, deg_a, deg_b, b2.reshape(1, D))


def _sc_degree(dst, zeros16, ones16):
    @functools.partial(
        pl.kernel,
        out_type=[jax.ShapeDtypeStruct((N, 16), jnp.float32),
                  jax.ShapeDtypeStruct((N, 16), jnp.float32)],
        mesh=_vector_mesh(),
        scratch_types=[
            pltpu.VMEM_SHARED((N_PAD, 16), jnp.float32),
            pltpu.VMEM((K,), jnp.int32),
            pltpu.VMEM((K, 16), jnp.float32),
            pltpu.SemaphoreType.DMA,
        ],
        compiler_params=_SC_PARAMS,
    )
    def deg_kernel(dst_hbm, z_hbm, ones_hbm, dega_hbm, degb_hbm,
                   deg_sh, idx_v, ones_v, sem):
        c = lax.axis_index("c")
        s = lax.axis_index("s")
        pltpu.sync_copy(z_hbm, deg_sh.at[pl.ds(s * RPT, RPT)])
        pltpu.sync_copy(ones_hbm, ones_v)
        plsc.subcore_barrier()
        base = (c * NS + s) * EPT

        @pl.loop(0, CH)
        def _(g):
            pltpu.sync_copy(dst_hbm.at[pl.ds(base + g * K, K)], idx_v)
            pltpu.sync_copy(ones_v, deg_sh.at[idx_v], add=True)

        plsc.subcore_barrier()

        @pl.when(c == 0)
        def _():
            pltpu.sync_copy(deg_sh.at[pl.ds(s * RPT, RPT)],
                            dega_hbm.at[pl.ds(s * RPT, RPT)])

        @pl.when(c == 1)
        def _():
            pltpu.sync_copy(deg_sh.at[pl.ds(s * RPT, RPT)],
                            degb_hbm.at[pl.ds(s * RPT, RPT)])

    return deg_kernel(dst, zeros16, ones16)


def _sc_gather_scatter(y, src, dst, zeros128):
    @functools.partial(
        pl.kernel,
        out_type=[jax.ShapeDtypeStruct((N, D), jnp.float32),
                  jax.ShapeDtypeStruct((N, D), jnp.float32)],
        mesh=_vector_mesh(),
        scratch_types=[
            pltpu.VMEM_SHARED((N_PAD, D), jnp.float32),
            pltpu.VMEM((K,), jnp.int32),
            pltpu.VMEM((K,), jnp.int32),
            pltpu.VMEM((K, D), jnp.float32),
            pltpu.SemaphoreType.DMA,
        ],
        compiler_params=_SC_PARAMS,
    )
    def gs_kernel(y_hbm, src_hbm, dst_hbm, z_hbm, acca_hbm, accb_hbm,
                  acc_sh, sidx_v, didx_v, buf_v, gsem):
        c = lax.axis_index("c")
        s = lax.axis_index("s")

        @pl.when(c == 0)
        def _():
            pltpu.sync_copy(y_hbm.at[pl.ds(s * RPT, RPT)],
                            acc_sh.at[pl.ds(s * RPT, RPT)])

        @pl.when(c == 1)
        def _():
            pltpu.sync_copy(z_hbm, acc_sh.at[pl.ds(s * RPT, RPT)])

        plsc.subcore_barrier()
        base = (c * NS + s) * EPT

        @pl.loop(0, CH)
        def _(g):
            off = base + g * K
            pltpu.sync_copy(src_hbm.at[pl.ds(off, K)], sidx_v)
            pltpu.sync_copy(dst_hbm.at[pl.ds(off, K)], didx_v)
            pltpu.async_copy(y_hbm.at[sidx_v], buf_v, gsem).wait()
            pltpu.sync_copy(buf_v, acc_sh.at[didx_v], add=True)

        plsc.subcore_barrier()

        @pl.when(c == 0)
        def _():
            pltpu.sync_copy(acc_sh.at[pl.ds(s * RPT, RPT)],
                            acca_hbm.at[pl.ds(s * RPT, RPT)])

        @pl.when(c == 1)
        def _():
            pltpu.sync_copy(acc_sh.at[pl.ds(s * RPT, RPT)],
                            accb_hbm.at[pl.ds(s * RPT, RPT)])

    return gs_kernel(y, src, dst, zeros128)


def kernel(x, edge_index, W1, b1, W2, b2):
    ei = edge_index.astype(jnp.int32)
    ppt = EPT - E // NW
    pad_src = jnp.zeros((NW, ppt), jnp.int32)
    pad_dst = jnp.broadcast_to(
        jnp.tile(jnp.arange(16, dtype=jnp.int32) + SINK, ppt // 16), (NW, ppt))
    src = jnp.concatenate([ei[0].reshape(NW, E // NW), pad_src], axis=1).reshape(-1)
    dst = jnp.concatenate([ei[1].reshape(NW, E // NW), pad_dst], axis=1).reshape(-1)
    zeros16 = jnp.zeros((RPT, 16), jnp.float32)
    ones16 = jnp.ones((K, 16), jnp.float32)
    zeros128 = jnp.zeros((RPT, D), jnp.float32)

    xw1 = _tc_matmul(x, W1)
    deg_a, deg_b = _sc_degree(dst, zeros16, ones16)
    y1 = _tc_scale(xw1, deg_a, deg_b)
    acc_a1, acc_b1 = _sc_gather_scatter(y1, src, dst, zeros128)
    y2 = _tc_mid(acc_a1, acc_b1, deg_a, deg_b, b1, W2)
    acc_a2, acc_b2 = _sc_gather_scatter(y2, src, dst, zeros128)
    return _tc_final(acc_a2, acc_b2, deg_a, deg_b, b2)


# speedup vs baseline: 9.9380x; 1.1767x over previous
import functools

import jax
import jax.numpy as jnp
from jax import lax
from jax.experimental import pallas as pl
from jax.experimental.pallas import tpu as pltpu
from jax.experimental.pallas import tpu_sc as plsc

N = 10000
D = 128
E = 320000
NC, NS = 2, 16
NW = NC * NS
K = 128
EPT = 10240
E_PAD = NW * EPT
CH = EPT // K
RPT = N // NS
N_PAD = N + 16
SINK = N
TB = 1000


def _vector_mesh():
    return plsc.VectorSubcoreMesh(core_axis_name="c", subcore_axis_name="s")


_SC_PARAMS = pltpu.CompilerParams(use_tc_tiling_on_sc=False)


def _dis_block(da_ref, db_ref):
    deg = da_ref[:, 0:1] + db_ref[:, 0:1] + 1.0
    return lax.rsqrt(deg)


def _tc_matmul(x, w):
    def body(x_ref, w_ref, o_ref):
        o_ref[...] = jnp.dot(x_ref[...], w_ref[...],
                             preferred_element_type=jnp.float32)
    return pl.pallas_call(
        body,
        grid=(N // TB,),
        in_specs=[pl.BlockSpec((TB, D), lambda i: (i, 0)),
                  pl.BlockSpec((D, D), lambda i: (0, 0))],
        out_specs=pl.BlockSpec((TB, D), lambda i: (i, 0)),
        out_shape=jax.ShapeDtypeStruct((N, D), jnp.float32),
    )(x, w)


def _tc_scale(xw, deg_a, deg_b):
    def body(x_ref, da_ref, db_ref, o_ref):
        o_ref[...] = x_ref[...] * _dis_block(da_ref, db_ref)
    return pl.pallas_call(
        body,
        grid=(N // TB,),
        in_specs=[pl.BlockSpec((TB, D), lambda i: (i, 0)),
                  pl.BlockSpec((TB, 16), lambda i: (i, 0)),
                  pl.BlockSpec((TB, 16), lambda i: (i, 0))],
        out_specs=pl.BlockSpec((TB, D), lambda i: (i, 0)),
        out_shape=jax.ShapeDtypeStruct((N, D), jnp.float32),
    )(xw, deg_a, deg_b)


def _tc_mid(acc_a, acc_b, deg_a, deg_b, b1, w2):
    def body(aa_ref, ab_ref, da_ref, db_ref, b_ref, w_ref, o_ref):
        dis = _dis_block(da_ref, db_ref)
        h = jnp.maximum(dis * (aa_ref[...] + ab_ref[...]) + b_ref[...], 0.0)
        o_ref[...] = jnp.dot(h, w_ref[...],
                             preferred_element_type=jnp.float32) * dis
    return pl.pallas_call(
        body,
        grid=(N // TB,),
        in_specs=[pl.BlockSpec((TB, D), lambda i: (i, 0)),
                  pl.BlockSpec((TB, D), lambda i: (i, 0)),
                  pl.BlockSpec((TB, 16), lambda i: (i, 0)),
                  pl.BlockSpec((TB, 16), lambda i: (i, 0)),
                  pl.BlockSpec((1, D), lambda i: (0, 0)),
                  pl.BlockSpec((D, D), lambda i: (0, 0))],
        out_specs=pl.BlockSpec((TB, D), lambda i: (i, 0)),
        out_shape=jax.ShapeDtypeStruct((N, D), jnp.float32),
    )(acc_a, acc_b, deg_a, deg_b, b1.reshape(1, D), w2)


def _tc_final(acc_a, acc_b, deg_a, deg_b, b2):
    def body(aa_ref, ab_ref, da_ref, db_ref, b_ref, o_ref):
        dis = _dis_block(da_ref, db_ref)
        o_ref[...] = dis * (aa_ref[...] + ab_ref[...]) + b_ref[...]
    return pl.pallas_call(
        body,
        grid=(N // TB,),
        in_specs=[pl.BlockSpec((TB, D), lambda i: (i, 0)),
                  pl.BlockSpec((TB, D), lambda i: (i, 0)),
                  pl.BlockSpec((TB, 16), lambda i: (i, 0)),
                  pl.BlockSpec((TB, 16), lambda i: (i, 0)),
                  pl.BlockSpec((1, D), lambda i: (0, 0))],
        out_specs=pl.BlockSpec((TB, D), lambda i: (i, 0)),
        out_shape=jax.ShapeDtypeStruct((N, D), jnp.float32),
    )(acc_a, acc_b, deg_a, deg_b, b2.reshape(1, D))


def _sc_degree(dst, zeros16, ones16):
    @functools.partial(
        pl.kernel,
        out_type=[jax.ShapeDtypeStruct((N, 16), jnp.float32),
                  jax.ShapeDtypeStruct((N, 16), jnp.float32)],
        mesh=_vector_mesh(),
        scratch_types=[
            pltpu.VMEM_SHARED((N_PAD, 16), jnp.float32),
            pltpu.VMEM((K,), jnp.int32),
            pltpu.VMEM((K, 16), jnp.float32),
            pltpu.SemaphoreType.DMA,
        ],
        compiler_params=_SC_PARAMS,
    )
    def deg_kernel(dst_hbm, z_hbm, ones_hbm, dega_hbm, degb_hbm,
                   deg_sh, idx_v, ones_v, sem):
        c = lax.axis_index("c")
        s = lax.axis_index("s")
        pltpu.sync_copy(z_hbm, deg_sh.at[pl.ds(s * RPT, RPT)])
        pltpu.sync_copy(ones_hbm, ones_v)
        plsc.subcore_barrier()
        base = (c * NS + s) * EPT

        @pl.loop(0, CH)
        def _(g):
            pltpu.sync_copy(dst_hbm.at[pl.ds(base + g * K, K)], idx_v)
            pltpu.sync_copy(ones_v, deg_sh.at[idx_v], add=True)

        plsc.subcore_barrier()

        @pl.when(c == 0)
        def _():
            pltpu.sync_copy(deg_sh.at[pl.ds(s * RPT, RPT)],
                            dega_hbm.at[pl.ds(s * RPT, RPT)])

        @pl.when(c == 1)
        def _():
            pltpu.sync_copy(deg_sh.at[pl.ds(s * RPT, RPT)],
                            degb_hbm.at[pl.ds(s * RPT, RPT)])

    return deg_kernel(dst, zeros16, ones16)


def _sc_gather_scatter(y, src, dst, zeros128):
    @functools.partial(
        pl.kernel,
        out_type=[jax.ShapeDtypeStruct((N, D), jnp.float32),
                  jax.ShapeDtypeStruct((N, D), jnp.float32)],
        mesh=_vector_mesh(),
        scratch_types=[
            pltpu.VMEM_SHARED((N_PAD, D), jnp.float32),
            pltpu.VMEM((K,), jnp.int32),
            pltpu.VMEM((K,), jnp.int32),
            pltpu.VMEM((K,), jnp.int32),
            pltpu.VMEM((K,), jnp.int32),
            pltpu.VMEM((K, D), jnp.float32),
            pltpu.VMEM((K, D), jnp.float32),
            pltpu.SemaphoreType.DMA,
            pltpu.SemaphoreType.DMA,
        ],
        compiler_params=_SC_PARAMS,
    )
    def gs_kernel(y_hbm, src_hbm, dst_hbm, z_hbm, acca_hbm, accb_hbm,
                  acc_sh, sidx0, didx0, sidx1, didx1, buf0, buf1,
                  gsem0, gsem1):
        c = lax.axis_index("c")
        s = lax.axis_index("s")

        @pl.when(c == 0)
        def _():
            pltpu.sync_copy(y_hbm.at[pl.ds(s * RPT, RPT)],
                            acc_sh.at[pl.ds(s * RPT, RPT)])

        @pl.when(c == 1)
        def _():
            pltpu.sync_copy(z_hbm, acc_sh.at[pl.ds(s * RPT, RPT)])

        plsc.subcore_barrier()
        base = (c * NS + s) * EPT
        sets = ((sidx0, didx0, buf0, gsem0), (sidx1, didx1, buf1, gsem1))

        def load_and_gather(chunk, st):
            si, di, bf, gs = st
            off = base + chunk * K
            pltpu.sync_copy(src_hbm.at[pl.ds(off, K)], si)
            pltpu.sync_copy(dst_hbm.at[pl.ds(off, K)], di)
            pltpu.async_copy(y_hbm.at[si], bf, gs)

        load_and_gather(0, sets[0])
        load_and_gather(1, sets[1])

        @pl.loop(0, CH // 2)
        def _(p):
            for j, st in enumerate(sets):
                si, di, bf, gs = st
                pltpu.make_async_copy(y_hbm.at[si], bf, gs).wait()
                pltpu.sync_copy(bf, acc_sh.at[di], add=True)

                @pl.when(p < CH // 2 - 1)
                def _():
                    load_and_gather(2 * p + 2 + j, st)

        plsc.subcore_barrier()

        @pl.when(c == 0)
        def _():
            pltpu.sync_copy(acc_sh.at[pl.ds(s * RPT, RPT)],
                            acca_hbm.at[pl.ds(s * RPT, RPT)])

        @pl.when(c == 1)
        def _():
            pltpu.sync_copy(acc_sh.at[pl.ds(s * RPT, RPT)],
                            accb_hbm.at[pl.ds(s * RPT, RPT)])

    return gs_kernel(y, src, dst, zeros128)


def kernel(x, edge_index, W1, b1, W2, b2):
    ei = edge_index.astype(jnp.int32)
    ppt = EPT - E // NW
    pad_src = jnp.zeros((NW, ppt), jnp.int32)
    pad_dst = jnp.broadcast_to(
        jnp.tile(jnp.arange(16, dtype=jnp.int32) + SINK, ppt // 16), (NW, ppt))
    src = jnp.concatenate([ei[0].reshape(NW, E // NW), pad_src], axis=1).reshape(-1)
    dst = jnp.concatenate([ei[1].reshape(NW, E // NW), pad_dst], axis=1).reshape(-1)
    zeros16 = jnp.zeros((RPT, 16), jnp.float32)
    ones16 = jnp.ones((K, 16), jnp.float32)
    zeros128 = jnp.zeros((RPT, D), jnp.float32)

    xw1 = _tc_matmul(x, W1)
    deg_a, deg_b = _sc_degree(dst, zeros16, ones16)
    y1 = _tc_scale(xw1, deg_a, deg_b)
    acc_a1, acc_b1 = _sc_gather_scatter(y1, src, dst, zeros128)
    y2 = _tc_mid(acc_a1, acc_b1, deg_a, deg_b, b1, W2)
    acc_a2, acc_b2 = _sc_gather_scatter(y2, src, dst, zeros128)
    return _tc_final(acc_a2, acc_b2, deg_a, deg_b, b2)
